# baseline (speedup 1.0000x reference)
import functools

import jax
import jax.numpy as jnp
from jax import lax
from jax.experimental import pallas as pl
from jax.experimental.pallas import tpu as pltpu
from jax.experimental.pallas import tpu_sc as plsc

N_FIELDS = 26
VOCAB = 100000
EMB = 64
BATCH = 4096

NC = 2
NS = 16
NW = NC * NS
BPW = BATCH // NW
NBUF = 6
NOB = 2

_mesh = plsc.VectorSubcoreMesh(core_axis_name="c", subcore_axis_name="s")


@functools.partial(
    pl.kernel,
    mesh=_mesh,
    out_type=jax.ShapeDtypeStruct((BATCH, N_FIELDS * EMB), jnp.float32),
    scratch_types=[
        pltpu.VMEM((BPW, N_FIELDS), jnp.int32),
        pltpu.VMEM((N_FIELDS, BPW), jnp.int32),
        pltpu.VMEM((NBUF, EMB, BPW), jnp.float32),
        pltpu.VMEM((NOB, BPW, EMB), jnp.float32),
        pltpu.SemaphoreType.DMA((NBUF,)),
        pltpu.SemaphoreType.DMA((NOB,)),
    ],
    compiler_params=pltpu.CompilerParams(
        use_tc_tiling_on_sc=False, needs_layout_passes=False
    ),
)
def _embed_gather(x_hbm, table_hbm, out_hbm, xv, idx_v, colbuf, rowbuf, gsem, osem):
    wid = lax.axis_index("s") * NC + lax.axis_index("c")
    b0 = wid * BPW

    pltpu.sync_copy(x_hbm.at[wid], xv)

    iota = lax.iota(jnp.int32, 16)

    @plsc.parallel_loop(0, N_FIELDS * (BPW // 16), 1, unroll=8)
    def _mkidx(t):
        f = t // (BPW // 16)
        j = t % (BPW // 16)
        rows = j * 16 + iota
        cols = jnp.full((16,), 0, jnp.int32) + f
        idx_v[f, pl.ds(j * 16, 16)] = plsc.load_gather(xv, [rows, cols])

    def gather(f, s):
        @plsc.parallel_loop(0, EMB, 1, unroll=4)
        def _g(d):
            pltpu.make_async_copy(
                table_hbm.at[f].at[d].at[idx_v.at[f]],
                colbuf.at[s].at[d],
                gsem.at[s],
            ).start()

    def gather_wait(f, s):
        pltpu.make_async_copy(
            table_hbm.at[f].at[:, pl.ds(0, BPW)],
            colbuf.at[s],
            gsem.at[s],
        ).wait()

    def transpose(s, o):
        @plsc.parallel_loop(0, BPW * (EMB // 16), 1, unroll=8)
        def _tp(t):
            k = t // (EMB // 16)
            j = t % (EMB // 16)
            rows = j * 16 + iota
            cols = jnp.full((16,), 0, jnp.int32) + k
            rowbuf[o, k, pl.ds(j * 16, 16)] = plsc.load_gather(
                colbuf.at[s], [rows, cols]
            )

    def copy_out(f, o):
        return pltpu.make_async_copy(
            rowbuf.at[o],
            out_hbm.at[pl.ds(b0, BPW), pl.ds(f * EMB, EMB)],
            osem.at[o],
        )

    for s in range(NBUF):
        gather(s, s)
    for f in range(N_FIELDS):
        s = f % NBUF
        o = f % NOB
        gather_wait(f, s)
        if f >= NOB:
            copy_out(f - NOB, o).wait()
        transpose(s, o)
        copy_out(f, o).start()
        if f + NBUF < N_FIELDS:
            gather(f + NBUF, s)
    for o in range(NOB):
        copy_out(N_FIELDS - NOB + o, o).wait()


def kernel(x, tables):
    x3 = x.astype(jnp.int32).reshape(NW, BPW, N_FIELDS)
    out = _embed_gather(x3, jnp.transpose(tables, (0, 2, 1)))
    return out


# speedup vs baseline: 1.0064x; 1.0064x over previous
import functools

import jax
import jax.numpy as jnp
from jax import lax
from jax.experimental import pallas as pl
from jax.experimental.pallas import tpu as pltpu
from jax.experimental.pallas import tpu_sc as plsc

N_FIELDS = 26
VOCAB = 100000
EMB = 64
BATCH = 4096

NC = 2
NS = 16
NW = NC * NS
BPW = BATCH // NW
NBUF = 4
NOB = 2

_mesh = plsc.VectorSubcoreMesh(core_axis_name="c", subcore_axis_name="s")


@functools.partial(
    pl.kernel,
    mesh=_mesh,
    out_type=jax.ShapeDtypeStruct((BATCH, N_FIELDS * EMB), jnp.float32),
    scratch_types=[
        pltpu.VMEM((BPW, N_FIELDS), jnp.int32),
        pltpu.VMEM((N_FIELDS, BPW), jnp.int32),
        pltpu.VMEM((NBUF, EMB, BPW), jnp.float32),
        pltpu.VMEM((NOB, BPW, EMB), jnp.float32),
        pltpu.SemaphoreType.DMA((NBUF,)),
        pltpu.SemaphoreType.DMA((NOB,)),
    ],
    compiler_params=pltpu.CompilerParams(
        use_tc_tiling_on_sc=False, needs_layout_passes=False
    ),
)
def _embed_gather(x_hbm, table_hbm, out_hbm, xv, idx_v, colbuf, rowbuf, gsem, osem):
    wid = lax.axis_index("s") * NC + lax.axis_index("c")
    b0 = wid * BPW

    pltpu.sync_copy(x_hbm.at[wid], xv)

    iota = lax.iota(jnp.int32, 16)

    @plsc.parallel_loop(0, N_FIELDS * (BPW // 16), 1, unroll=8)
    def _mkidx(t):
        f = t // (BPW // 16)
        j = t % (BPW // 16)
        rows = j * 16 + iota
        cols = jnp.full((16,), 0, jnp.int32) + f
        idx_v[f, pl.ds(j * 16, 16)] = plsc.load_gather(xv, [rows, cols])

    def gather(f, s):
        def body(d, carry):
            pltpu.make_async_copy(
                table_hbm.at[f].at[d].at[idx_v.at[f]],
                colbuf.at[s].at[d],
                gsem.at[s],
            ).start()
            return carry

        lax.fori_loop(0, EMB, body, 0)

    def gather_wait(f, s):
        pltpu.make_async_copy(
            table_hbm.at[f].at[:, pl.ds(0, BPW)],
            colbuf.at[s],
            gsem.at[s],
        ).wait()

    def transpose(s, o):
        @plsc.parallel_loop(0, BPW * (EMB // 16), 1, unroll=8)
        def _tp(t):
            k = t // (EMB // 16)
            j = t % (EMB // 16)
            rows = j * 16 + iota
            cols = jnp.full((16,), 0, jnp.int32) + k
            rowbuf[o, k, pl.ds(j * 16, 16)] = plsc.load_gather(
                colbuf.at[s], [rows, cols]
            )

    def copy_out(f, o):
        return pltpu.make_async_copy(
            rowbuf.at[o],
            out_hbm.at[pl.ds(b0, BPW), pl.ds(f * EMB, EMB)],
            osem.at[o],
        )

    for s in range(NBUF):
        gather(s, s)
    for f in range(N_FIELDS):
        s = f % NBUF
        o = f % NOB
        gather_wait(f, s)
        if f >= NOB:
            copy_out(f - NOB, o).wait()
        transpose(s, o)
        copy_out(f, o).start()
        if f + NBUF < N_FIELDS:
            gather(f + NBUF, s)
    for o in range(NOB):
        copy_out(N_FIELDS - NOB + o, o).wait()


def kernel(x, tables):
    x3 = x.astype(jnp.int32).reshape(NW, BPW, N_FIELDS)
    out = _embed_gather(x3, jnp.transpose(tables, (0, 2, 1)))
    return out
